# baseline (speedup 1.0000x reference)
import functools

import jax
import jax.numpy as jnp
import numpy as np
from jax import lax
from jax.experimental import pallas as pl
from jax.experimental.pallas import tpu as pltpu
from jax.experimental.pallas import tpu_sc as plsc

_B = 32
_N = 8192
_D = 64
_K = 64
_TK = 512
_NCH = _N // _TK
_NC = 2
_UNROLL = 1
_SBIG = 1e38

_U32 = np.uint32
_ROT1 = (13, 15, 26, 6)
_ROT2 = (17, 29, 16, 24)


def _tf_rounds(x0, x1, rots):
    for r in rots:
        x0 = (x0 + x1).astype(_U32)
        x1 = (np.left_shift(x1, _U32(r))
              | np.right_shift(x1, _U32(32 - r))).astype(_U32)
        x1 = x0 ^ x1
    return x0, x1


def _tf_cipher(k1, k2, x0, x1):
    ks2 = (k1 ^ k2 ^ _U32(0x1BD11BDA)).astype(_U32)
    x0 = (np.asarray(x0, _U32) + k1).astype(_U32)
    x1 = (np.asarray(x1, _U32) + k2).astype(_U32)
    x0, x1 = _tf_rounds(x0, x1, _ROT1)
    x0 = (x0 + k2).astype(_U32); x1 = (x1 + ks2 + _U32(1)).astype(_U32)
    x0, x1 = _tf_rounds(x0, x1, _ROT2)
    x0 = (x0 + ks2).astype(_U32); x1 = (x1 + k1 + _U32(2)).astype(_U32)
    x0, x1 = _tf_rounds(x0, x1, _ROT1)
    x0 = (x0 + k1).astype(_U32); x1 = (x1 + k2 + _U32(3)).astype(_U32)
    x0, x1 = _tf_rounds(x0, x1, _ROT2)
    x0 = (x0 + k2).astype(_U32); x1 = (x1 + ks2 + _U32(4)).astype(_U32)
    x0, x1 = _tf_rounds(x0, x1, _ROT1)
    x0 = (x0 + ks2).astype(_U32); x1 = (x1 + k1 + _U32(5)).astype(_U32)
    return x0, x1


def _tf_split(key, n):
    lo = np.arange(n, dtype=_U32)
    hi = np.zeros(n, _U32)
    b1, b2 = _tf_cipher(key[..., 0, None], key[..., 1, None], hi, lo)
    return np.stack([b1, b2], -1)


def _tf_bits32(key, n):
    lo = np.arange(n, dtype=_U32)
    hi = np.zeros(n, _U32)
    b1, b2 = _tf_cipher(key[..., 0, None], key[..., 1, None], hi, lo)
    return b1 ^ b2


def _tf_gumbel(key, n):
    bits = _tf_bits32(key, n)
    fb = (np.right_shift(bits, _U32(9)) | _U32(0x3F800000)).astype(_U32)
    f = fb.view(np.float32) - np.float32(1.0)
    tiny = np.float32(np.finfo(np.float32).tiny)
    u = np.maximum(tiny, (f * (np.float32(1.0) - tiny) + tiny))
    return -np.log(-np.log(u))


def _sampling_weights():
    keys = _tf_split(np.array([0, 42], _U32), _B)
    ks = _tf_split(keys, 2)
    k, sub = ks[:, 0], ks[:, 1]
    first = (_tf_bits32(_tf_split(sub, 2)[:, 1], 1)[:, 0] % _U32(_N))
    w = np.zeros((_B, _K, _N), np.float32)
    w[np.arange(_B), 0, first] = 1.0
    for t in range(1, _K):
        ks = _tf_split(k, 2)
        k, sub = ks[:, 0], ks[:, 1]
        w[:, t] = np.exp(np.float32(2.0) * _tf_gumbel(sub, _N))
    return w


_W_TABLE = _sampling_weights()


def _chunk_copy(xt_hbm, b, ci, buf, sem):
    return pltpu.make_async_copy(
        xt_hbm.at[b, :, pl.ds(ci * _TK, _TK)], buf, sem)


def _broadcast16(cv):
    return [jnp.full((16,), cv[dd], jnp.float32) for dd in range(16)]


def _process_chunk(buf, chunk_base, c_ref, s_ref, w_ref, acc_ref, scan):
    mv, iv = scan
    for db in range(4):
        cb = _broadcast16(c_ref[pl.ds(db * 16, 16)])

        def gbody(g, carry, db=db, cb=cb):
            mv, iv = carry
            for u in range(_UNROLL):
                base = (g * _UNROLL + u) * 16
                a = [jnp.zeros((16,), jnp.float32) for _ in range(4)]
                for dd in range(16):
                    xv = buf[db * 16 + dd, pl.ds(base, 16)]
                    d_ = xv - cb[dd]
                    a[dd % 4] = a[dd % 4] + d_ * d_
                acc = (a[0] + a[1]) + (a[2] + a[3])
                if db > 0:
                    acc = acc + acc_ref[pl.ds(base, 16)]
                if db < 3:
                    acc_ref[pl.ds(base, 16)] = acc
                else:
                    so = chunk_base + base
                    s = jnp.minimum(s_ref[pl.ds(so, 16)], acc)
                    s_ref[pl.ds(so, 16)] = s
                    p = s * w_ref[pl.ds(so, 16)]
                    upd = p > mv
                    mv = jnp.where(upd, p, mv)
                    iv = jnp.where(upd, so + _LANE, iv)
            return mv, iv

        mv, iv = plsc.parallel_loop(
            0, _TK // 16 // _UNROLL, 1, unroll=4, carry=(mv, iv))(gbody)
    return mv, iv


_LANE = None


def _scan_to_index(mv, iv):
    m = jnp.max(mv)
    return jnp.min(jnp.where(mv == m, iv, jnp.int32(2 ** 30)))


def _body(x_hbm, xt_hbm, w_hbm, out_hbm,
          s_ref, w_ref, xa_ref, xb_ref, acc_ref, c_ref,
          sem_a, sem_b, sem_w):
    global _LANE
    b = lax.axis_index("s") * _NC + lax.axis_index("c")
    _LANE = lax.iota(jnp.int32, 16)

    big16 = jnp.full((16,), _SBIG, jnp.float32)

    def init_body(g, _):
        s_ref[pl.ds(g * 16, 16)] = big16
        return 0
    lax.fori_loop(0, _N // 16, init_body, 0)

    _chunk_copy(xt_hbm, b, 0, xa_ref, sem_a).start()

    pltpu.sync_copy(w_hbm.at[b, 0], w_ref)

    def abody(g, carry):
        mv, iv = carry
        p = s_ref[pl.ds(g * 16, 16)] * w_ref[pl.ds(g * 16, 16)]
        upd = p > mv
        return (jnp.where(upd, p, mv),
                jnp.where(upd, g * 16 + _LANE, iv))

    mv0, iv0 = lax.fori_loop(
        0, _N // 16, abody,
        (jnp.full((16,), -1.0, jnp.float32), jnp.zeros((16,), jnp.int32)))
    idx0 = _scan_to_index(mv0, iv0)

    def step(t, idx):
        pltpu.make_async_copy(w_hbm.at[b, t + 1], w_ref, sem_w).start()
        pltpu.sync_copy(x_hbm.at[b, idx], c_ref)
        pltpu.sync_copy(c_ref, out_hbm.at[b, t])
        pltpu.make_async_copy(w_hbm.at[b, t + 1], w_ref, sem_w).wait()

        def pair(g, carry):
            c0 = 2 * g
            c1 = 2 * g + 1
            nxt = (2 * g + 2) % _NCH
            _chunk_copy(xt_hbm, b, c1, xb_ref, sem_b).start()
            _chunk_copy(xt_hbm, b, c0, xa_ref, sem_a).wait()
            carry = _process_chunk(
                xa_ref, c0 * _TK, c_ref, s_ref, w_ref, acc_ref, carry)
            _chunk_copy(xt_hbm, b, nxt, xa_ref, sem_a).start()
            _chunk_copy(xt_hbm, b, c1, xb_ref, sem_b).wait()
            return _process_chunk(
                xb_ref, c1 * _TK, c_ref, s_ref, w_ref, acc_ref, carry)

        mv, iv = lax.fori_loop(
            0, _NCH // 2, pair,
            (jnp.full((16,), -1.0, jnp.float32), jnp.zeros((16,), jnp.int32)))
        return _scan_to_index(mv, iv)

    idx_last = lax.fori_loop(0, _K - 1, step, idx0)

    pltpu.sync_copy(x_hbm.at[b, idx_last], c_ref)
    pltpu.sync_copy(c_ref, out_hbm.at[b, _K - 1])

    _chunk_copy(xt_hbm, b, 0, xa_ref, sem_a).wait()


_KERNEL_CACHE = []


def _diverse_sc():
    if _KERNEL_CACHE:
        return _KERNEL_CACHE[0]
    f = functools.partial(
        pl.kernel,
        mesh=plsc.VectorSubcoreMesh(core_axis_name="c", subcore_axis_name="s"),
        compiler_params=pltpu.CompilerParams(needs_layout_passes=False),
        out_type=jax.ShapeDtypeStruct((_B, _K, _D), jnp.float32),
        scratch_types=[
            pltpu.VMEM((_N,), jnp.float32),
            pltpu.VMEM((_N,), jnp.float32),
            pltpu.VMEM((_D, _TK), jnp.float32),
            pltpu.VMEM((_D, _TK), jnp.float32),
            pltpu.VMEM((_TK,), jnp.float32),
            pltpu.VMEM((_D,), jnp.float32),
            pltpu.SemaphoreType.DMA,
            pltpu.SemaphoreType.DMA,
            pltpu.SemaphoreType.DMA,
        ],
    )(_body)
    _KERNEL_CACHE.append(f)
    return f


def kernel(x):
    w = jnp.asarray(_W_TABLE)
    xt = jnp.swapaxes(x, 1, 2)
    tokens = _diverse_sc()(x, xt, w)
    return tokens, jnp.float32(0.0)


# speedup vs baseline: 1.3228x; 1.3228x over previous
import functools

import jax
import jax.numpy as jnp
import numpy as np
from jax import lax
from jax.experimental import pallas as pl
from jax.experimental.pallas import tpu as pltpu
from jax.experimental.pallas import tpu_sc as plsc

_B = 32
_N = 8192
_D = 64
_K = 64
_TK = 512
_NCH = _N // _TK
_NC = 2
_UNROLL = 1
_SBIG = 1e38

_U32 = np.uint32
_ROT1 = (13, 15, 26, 6)
_ROT2 = (17, 29, 16, 24)


def _tf_rounds(x0, x1, rots):
    for r in rots:
        x0 = (x0 + x1).astype(_U32)
        x1 = (np.left_shift(x1, _U32(r))
              | np.right_shift(x1, _U32(32 - r))).astype(_U32)
        x1 = x0 ^ x1
    return x0, x1


def _tf_cipher(k1, k2, x0, x1):
    ks2 = (k1 ^ k2 ^ _U32(0x1BD11BDA)).astype(_U32)
    x0 = (np.asarray(x0, _U32) + k1).astype(_U32)
    x1 = (np.asarray(x1, _U32) + k2).astype(_U32)
    x0, x1 = _tf_rounds(x0, x1, _ROT1)
    x0 = (x0 + k2).astype(_U32); x1 = (x1 + ks2 + _U32(1)).astype(_U32)
    x0, x1 = _tf_rounds(x0, x1, _ROT2)
    x0 = (x0 + ks2).astype(_U32); x1 = (x1 + k1 + _U32(2)).astype(_U32)
    x0, x1 = _tf_rounds(x0, x1, _ROT1)
    x0 = (x0 + k1).astype(_U32); x1 = (x1 + k2 + _U32(3)).astype(_U32)
    x0, x1 = _tf_rounds(x0, x1, _ROT2)
    x0 = (x0 + k2).astype(_U32); x1 = (x1 + ks2 + _U32(4)).astype(_U32)
    x0, x1 = _tf_rounds(x0, x1, _ROT1)
    x0 = (x0 + ks2).astype(_U32); x1 = (x1 + k1 + _U32(5)).astype(_U32)
    return x0, x1


def _tf_split(key, n):
    lo = np.arange(n, dtype=_U32)
    hi = np.zeros(n, _U32)
    b1, b2 = _tf_cipher(key[..., 0, None], key[..., 1, None], hi, lo)
    return np.stack([b1, b2], -1)


def _tf_bits32(key, n):
    lo = np.arange(n, dtype=_U32)
    hi = np.zeros(n, _U32)
    b1, b2 = _tf_cipher(key[..., 0, None], key[..., 1, None], hi, lo)
    return b1 ^ b2


def _tf_gumbel(key, n):
    bits = _tf_bits32(key, n)
    fb = (np.right_shift(bits, _U32(9)) | _U32(0x3F800000)).astype(_U32)
    f = fb.view(np.float32) - np.float32(1.0)
    tiny = np.float32(np.finfo(np.float32).tiny)
    u = np.maximum(tiny, (f * (np.float32(1.0) - tiny) + tiny))
    return -np.log(-np.log(u))


def _sampling_weights():
    keys = _tf_split(np.array([0, 42], _U32), _B)
    ks = _tf_split(keys, 2)
    k, sub = ks[:, 0], ks[:, 1]
    first = (_tf_bits32(_tf_split(sub, 2)[:, 1], 1)[:, 0] % _U32(_N))
    w = np.zeros((_B, _K, _N), np.float32)
    w[np.arange(_B), 0, first] = 1.0
    for t in range(1, _K):
        ks = _tf_split(k, 2)
        k, sub = ks[:, 0], ks[:, 1]
        w[:, t] = np.exp(np.float32(2.0) * _tf_gumbel(sub, _N))
    return w


_W_TABLE = _sampling_weights()


def _chunk_copy(xt_hbm, b, ci, buf, sem):
    return pltpu.make_async_copy(
        xt_hbm.at[b, :, pl.ds(ci * _TK, _TK)], buf, sem)


def _broadcast16(cv):
    return [jnp.full((16,), cv[dd], jnp.float32) for dd in range(16)]


def _process_chunk(buf, chunk_base, c_ref, s_ref, w_ref, acc_ref, scan):
    mv, iv = scan
    for db in range(4):
        cb = _broadcast16(c_ref[pl.ds(db * 16, 16)])

        def gbody(g, carry, db=db, cb=cb):
            mv, iv = carry
            for u in range(_UNROLL):
                base = (g * _UNROLL + u) * 16
                a = [jnp.zeros((16,), jnp.float32) for _ in range(4)]
                for dd in range(16):
                    xv = buf[db * 16 + dd, pl.ds(base, 16)]
                    d_ = xv - cb[dd]
                    a[dd % 4] = a[dd % 4] + d_ * d_
                acc = (a[0] + a[1]) + (a[2] + a[3])
                if db > 0:
                    acc = acc + acc_ref[pl.ds(base, 16)]
                if db < 3:
                    acc_ref[pl.ds(base, 16)] = acc
                else:
                    so = chunk_base + base
                    s = jnp.minimum(s_ref[pl.ds(so, 16)], acc)
                    s_ref[pl.ds(so, 16)] = s
                    p = s * w_ref[pl.ds(so, 16)]
                    upd = p > mv
                    mv = jnp.where(upd, p, mv)
                    iv = jnp.where(upd, so + _LANE, iv)
            return mv, iv

        mv, iv = plsc.parallel_loop(
            0, _TK // 16 // _UNROLL, 1, unroll=2, carry=(mv, iv))(gbody)
    return mv, iv


_LANE = None


def _scan_to_index(mv, iv):
    m = jnp.max(mv)
    return jnp.min(jnp.where(mv == m, iv, jnp.int32(2 ** 30)))


def _body(x_hbm, xt_hbm, w_hbm, out_hbm,
          s_ref, w_ref, xa_ref, xb_ref, acc_ref, c_ref,
          sem_a, sem_b, sem_w):
    global _LANE
    b = lax.axis_index("s") * _NC + lax.axis_index("c")
    _LANE = lax.iota(jnp.int32, 16)

    big16 = jnp.full((16,), _SBIG, jnp.float32)

    def init_body(g, _):
        s_ref[pl.ds(g * 16, 16)] = big16
        return 0
    lax.fori_loop(0, _N // 16, init_body, 0)

    _chunk_copy(xt_hbm, b, 0, xa_ref, sem_a).start()

    pltpu.sync_copy(w_hbm.at[b, 0], w_ref)

    def abody(g, carry):
        mv, iv = carry
        p = s_ref[pl.ds(g * 16, 16)] * w_ref[pl.ds(g * 16, 16)]
        upd = p > mv
        return (jnp.where(upd, p, mv),
                jnp.where(upd, g * 16 + _LANE, iv))

    mv0, iv0 = lax.fori_loop(
        0, _N // 16, abody,
        (jnp.full((16,), -1.0, jnp.float32), jnp.zeros((16,), jnp.int32)))
    idx0 = _scan_to_index(mv0, iv0)

    def step(t, idx):
        pltpu.make_async_copy(w_hbm.at[b, t + 1], w_ref, sem_w).start()
        pltpu.sync_copy(x_hbm.at[b, idx], c_ref)
        pltpu.sync_copy(c_ref, out_hbm.at[b, t])
        pltpu.make_async_copy(w_hbm.at[b, t + 1], w_ref, sem_w).wait()

        def pair(g, carry):
            c0 = 2 * g
            c1 = 2 * g + 1
            nxt = (2 * g + 2) % _NCH
            _chunk_copy(xt_hbm, b, c1, xb_ref, sem_b).start()
            _chunk_copy(xt_hbm, b, c0, xa_ref, sem_a).wait()
            carry = _process_chunk(
                xa_ref, c0 * _TK, c_ref, s_ref, w_ref, acc_ref, carry)
            _chunk_copy(xt_hbm, b, nxt, xa_ref, sem_a).start()
            _chunk_copy(xt_hbm, b, c1, xb_ref, sem_b).wait()
            return _process_chunk(
                xb_ref, c1 * _TK, c_ref, s_ref, w_ref, acc_ref, carry)

        mv, iv = lax.fori_loop(
            0, _NCH // 2, pair,
            (jnp.full((16,), -1.0, jnp.float32), jnp.zeros((16,), jnp.int32)))
        return _scan_to_index(mv, iv)

    idx_last = lax.fori_loop(0, _K - 1, step, idx0)

    pltpu.sync_copy(x_hbm.at[b, idx_last], c_ref)
    pltpu.sync_copy(c_ref, out_hbm.at[b, _K - 1])

    _chunk_copy(xt_hbm, b, 0, xa_ref, sem_a).wait()


_KERNEL_CACHE = []


def _diverse_sc():
    if _KERNEL_CACHE:
        return _KERNEL_CACHE[0]
    f = functools.partial(
        pl.kernel,
        mesh=plsc.VectorSubcoreMesh(core_axis_name="c", subcore_axis_name="s"),
        compiler_params=pltpu.CompilerParams(needs_layout_passes=False),
        out_type=jax.ShapeDtypeStruct((_B, _K, _D), jnp.float32),
        scratch_types=[
            pltpu.VMEM((_N,), jnp.float32),
            pltpu.VMEM((_N,), jnp.float32),
            pltpu.VMEM((_D, _TK), jnp.float32),
            pltpu.VMEM((_D, _TK), jnp.float32),
            pltpu.VMEM((_TK,), jnp.float32),
            pltpu.VMEM((_D,), jnp.float32),
            pltpu.SemaphoreType.DMA,
            pltpu.SemaphoreType.DMA,
            pltpu.SemaphoreType.DMA,
        ],
    )(_body)
    _KERNEL_CACHE.append(f)
    return f


def kernel(x):
    w = jnp.asarray(_W_TABLE)
    xt = jnp.swapaxes(x, 1, 2)
    tokens = _diverse_sc()(x, xt, w)
    return tokens, jnp.float32(0.0)
